# Initial kernel scaffold; baseline (speedup 1.0000x reference)
import jax
import jax.numpy as jnp
from jax.experimental import pallas as pl


def kernel(x, edge_index, W, b):
    raise NotImplementedError("write your pallas kernel here")


# speedup vs baseline: 9.3875x; 9.3875x over previous
import functools

import jax
import jax.numpy as jnp
from jax import lax
from jax.experimental import pallas as pl
from jax.experimental.pallas import tpu as pltpu
from jax.experimental.pallas import tpu_sc as plsc

NC = 2
NS = 16
NW = NC * NS

SUB = 128
K = 2
CH = SUB * K
GQ = 5
EQ = CH * GQ
NQ = 8
NPAD = 10240
EPW = EQ * NQ
EPAD = NW * EPW
ZR = 128


def _matmul_body(x_ref, w_ref, b_ref, o_ref):
    o_ref[...] = lax.dot_general(
        x_ref[...], w_ref[...],
        (((1,), (1,)), ((), ())),
        preferred_element_type=jnp.float32,
    ) + b_ref[...]


def _add3_body(a_ref, b_ref, c_ref, o_ref):
    o_ref[...] = a_ref[...] + b_ref[...] + c_ref[...]


def _make_sc_agg(N, D):
    RT = NPAD // NS

    def body(out_hbm, row_hbm, col_hbm, agg0, agg1,
             acc, msgs, row_a, col_a, row_b, col_b, sg0, sg1, ss0, ss1):
        sem_g = [sg0, sg1]
        sem_s = [ss0, ss1]
        row_bufs = [row_a, row_b]
        col_bufs = [col_a, col_b]
        c = lax.axis_index("c")
        s = lax.axis_index("s")
        w = c * NS + s
        wbase = w * EPW

        def zvec(i, _):
            r = i // (D // 16)
            j = i % (D // 16)
            msgs[r, pl.ds(j * 16, 16)] = jnp.zeros((16,), jnp.float32)
            return 0
        lax.fori_loop(0, CH * (D // 16), zvec, 0)

        def zcopy(i, _):
            pltpu.sync_copy(msgs.at[pl.ds(0, ZR)],
                            acc.at[pl.ds(s * RT + i * ZR, ZR)])
            return 0
        lax.fori_loop(0, RT // ZR, zcopy, 0)
        plsc.subcore_barrier()

        pltpu.sync_copy(row_hbm.at[pl.ds(wbase, EQ)], row_b)
        for j in range(K):
            pltpu.async_copy(msgs.at[pl.ds(j * SUB, SUB)],
                             acc.at[row_b.at[pl.ds(j * SUB, SUB)]],
                             sem_s[j], add=True)

        def pair(qq, _):
            for p in range(2):
                q = qq * 2 + p
                row_big = row_bufs[p]
                col_big = col_bufs[p]
                ebase = wbase + q * EQ
                pltpu.sync_copy(row_hbm.at[pl.ds(ebase, EQ)], row_big)
                pltpu.sync_copy(col_hbm.at[pl.ds(ebase, EQ)], col_big)

                def group(g, _):
                    cps = []
                    for j in range(K):
                        off = (g * K + j) * SUB
                        pltpu.make_async_copy(
                            msgs.at[pl.ds(j * SUB, SUB)],
                            acc.at[row_big.at[pl.ds(off, SUB)]],
                            sem_s[j]).wait()
                        cps.append(pltpu.async_copy(
                            out_hbm.at[col_big.at[pl.ds(off, SUB)]],
                            msgs.at[pl.ds(j * SUB, SUB)], sem_g[j]))
                    for j in range(K):
                        off = (g * K + j) * SUB
                        cps[j].wait()
                        pltpu.async_copy(msgs.at[pl.ds(j * SUB, SUB)],
                                         acc.at[row_big.at[pl.ds(off, SUB)]],
                                         sem_s[j], add=True)
                    return 0
                lax.fori_loop(0, GQ, group, 0)
            return 0
        lax.fori_loop(0, NQ // 2, pair, 0)

        for j in range(K):
            pltpu.make_async_copy(msgs.at[pl.ds(j * SUB, SUB)],
                                  acc.at[row_b.at[pl.ds(j * SUB, SUB)]],
                                  sem_s[j]).wait()
        plsc.subcore_barrier()

        @pl.when(c == 0)
        def _():
            pltpu.sync_copy(acc.at[pl.ds(s * RT, RT)],
                            agg0.at[pl.ds(s * RT, RT)])

        @pl.when(c == 1)
        def _():
            pltpu.sync_copy(acc.at[pl.ds(s * RT, RT)],
                            agg1.at[pl.ds(s * RT, RT)])

    mesh = plsc.VectorSubcoreMesh(
        core_axis_name="c", subcore_axis_name="s",
        num_cores=NC, num_subcores=NS)
    return pl.kernel(
        body,
        out_type=(
            jax.ShapeDtypeStruct((NPAD, D), jnp.float32),
            jax.ShapeDtypeStruct((NPAD, D), jnp.float32),
        ),
        mesh=mesh,
        scratch_types=[
            pltpu.VMEM_SHARED((NPAD, D), jnp.float32),
            pltpu.VMEM((CH, D), jnp.float32),
            pltpu.VMEM((EQ,), jnp.int32),
            pltpu.VMEM((EQ,), jnp.int32),
            pltpu.VMEM((EQ,), jnp.int32),
            pltpu.VMEM((EQ,), jnp.int32),
            pltpu.SemaphoreType.DMA,
            pltpu.SemaphoreType.DMA,
            pltpu.SemaphoreType.DMA,
            pltpu.SemaphoreType.DMA,
        ],
        name="sc_graph_agg",
    )


@jax.jit
def kernel(x, edge_index, W, b):
    N, D_in = x.shape
    D = W.shape[0]
    E = edge_index.shape[1]
    assert EPAD >= E and NPAD % (NS * ZR) == 0

    BM = 1000
    out = pl.pallas_call(
        _matmul_body,
        grid=(N // BM,),
        in_specs=[
            pl.BlockSpec((BM, D_in), lambda i: (i, 0)),
            pl.BlockSpec((D, D_in), lambda i: (0, 0)),
            pl.BlockSpec((1, D), lambda i: (0, 0)),
        ],
        out_specs=pl.BlockSpec((BM, D), lambda i: (i, 0)),
        out_shape=jax.ShapeDtypeStruct((N, D), jnp.float32),
    )(x, W, b.reshape(1, D))

    npad_e = EPAD - E
    rpad = N + (jnp.arange(npad_e, dtype=jnp.int32) % (NPAD - N))
    cpad = jnp.arange(npad_e, dtype=jnp.int32) % N
    row = jnp.concatenate([edge_index[0], rpad])
    col = jnp.concatenate([edge_index[1], cpad])

    agg0, agg1 = _make_sc_agg(N, D)(out, row, col)

    final = pl.pallas_call(
        _add3_body,
        grid=(N // BM,),
        in_specs=[pl.BlockSpec((BM, D), lambda i: (i, 0))] * 3,
        out_specs=pl.BlockSpec((BM, D), lambda i: (i, 0)),
        out_shape=jax.ShapeDtypeStruct((N, D), jnp.float32),
    )(out, agg0, agg1)
    return final
